# speedup vs baseline: 1.3243x; 1.3243x over previous
import functools

import jax
import jax.numpy as jnp
from jax import lax
from jax.experimental import pallas as pl
from jax.experimental.pallas import tpu as pltpu
from jax.experimental.pallas import tpu_sc as plsc

N = 10000
DEG = 32
DIN = 128
DOUT = 128

NC = 2
NS = 16
L = 16
NW = NC * NS
NPW = 320
NPAD = NW * NPW
NB = 8
ROWS = NB * DEG
NBATCH = NPW // NB


def _mm_body(f_ref, w_ref, b_ref, o_ref):
    y = jnp.dot(f_ref[...], w_ref[...], preferred_element_type=jnp.float32)
    o_ref[...] = jnp.maximum(y + b_ref[...], 0.0) * (1.0 / DEG)


def _linear_relu_scaled(feats_pad, W, b2):
    blk = 2560
    return pl.pallas_call(
        _mm_body,
        grid=(NPAD // blk,),
        in_specs=[
            pl.BlockSpec((blk, DIN), lambda i: (i, 0)),
            pl.BlockSpec((DIN, DOUT), lambda i: (0, 0)),
            pl.BlockSpec((1, DOUT), lambda i: (0, 0)),
        ],
        out_specs=pl.BlockSpec((blk, DOUT), lambda i: (i, 0)),
        out_shape=jax.ShapeDtypeStruct((NPAD, DOUT), jnp.float32),
    )(feats_pad, W, b2)


def _sc_body(y_hbm, eidx_hbm, out_hbm, idx_v, rows_v, acc_v, sem):
    c = lax.axis_index("c")
    s = lax.axis_index("s")
    wid = s * NC + c
    base = wid * NPW

    pltpu.sync_copy(eidx_hbm.at[pl.ds(base * DEG, NPW * DEG)], idx_v)

    def batch_body(bi, carry):
        pltpu.async_copy(
            y_hbm.at[idx_v.at[pl.ds(bi * ROWS, ROWS)]], rows_v, sem
        ).wait()

        def node_body(ni, carry2):
            r0 = ni * DEG
            for ch in range(DOUT // L):
                col = pl.ds(ch * L, L)
                p0 = rows_v[r0 + 0, col]
                p1 = rows_v[r0 + 1, col]
                p2 = rows_v[r0 + 2, col]
                p3 = rows_v[r0 + 3, col]
                for j in range(4, DEG, 4):
                    p0 = p0 + rows_v[r0 + j + 0, col]
                    p1 = p1 + rows_v[r0 + j + 1, col]
                    p2 = p2 + rows_v[r0 + j + 2, col]
                    p3 = p3 + rows_v[r0 + j + 3, col]
                acc_v[bi * NB + ni, col] = (p0 + p1) + (p2 + p3)
            return carry2

        lax.fori_loop(0, NB, node_body, 0)
        return carry

    lax.fori_loop(0, NBATCH, batch_body, 0)

    pltpu.sync_copy(acc_v, out_hbm.at[pl.ds(base, NPW)])


def _sc_gather(y, eidx):
    mesh = plsc.VectorSubcoreMesh(core_axis_name="c", subcore_axis_name="s")
    fn = functools.partial(
        pl.kernel,
        mesh=mesh,
        out_type=jax.ShapeDtypeStruct((NPAD, DOUT), jnp.float32),
        scratch_types=[
            pltpu.VMEM((NPW * DEG,), jnp.int32),
            pltpu.VMEM((ROWS, DOUT), jnp.float32),
            pltpu.VMEM((NPW, DOUT), jnp.float32),
            pltpu.SemaphoreType.DMA,
        ],
    )(_sc_body)
    return fn(y, eidx)


@jax.jit
def _impl(feats, edge_dict, W, b):
    feats_pad = jnp.pad(feats, ((0, NPAD - N), (0, 0)))
    y = _linear_relu_scaled(feats_pad, W, b.reshape(1, DOUT))
    eidx = jnp.pad(edge_dict, ((0, NPAD - N), (0, 0))).reshape(-1)
    pooled = _sc_gather(y, eidx)
    return pooled[:N]


def kernel(ids, feats, edge_dict, G, ite, W, b):
    return _impl(feats, edge_dict, W, b)


# speedup vs baseline: 1.4623x; 1.1042x over previous
import functools

import jax
import jax.numpy as jnp
from jax import lax
from jax.experimental import pallas as pl
from jax.experimental.pallas import tpu as pltpu
from jax.experimental.pallas import tpu_sc as plsc

N = 10000
DEG = 32
DIN = 128
DOUT = 128

NC = 2
NS = 16
L = 16
NW = NC * NS
NPW = 320
NPAD = NW * NPW
NB = 8
ROWS = NB * DEG
NBATCH = NPW // NB


def _mm_body(f_ref, w_ref, b_ref, o_ref):
    y = jnp.dot(f_ref[...], w_ref[...], preferred_element_type=jnp.float32)
    o_ref[...] = jnp.maximum(y + b_ref[...], 0.0) * (1.0 / DEG)


def _linear_relu_scaled(feats_pad, W, b2):
    blk = 2560
    return pl.pallas_call(
        _mm_body,
        grid=(NPAD // blk,),
        in_specs=[
            pl.BlockSpec((blk, DIN), lambda i: (i, 0)),
            pl.BlockSpec((DIN, DOUT), lambda i: (0, 0)),
            pl.BlockSpec((1, DOUT), lambda i: (0, 0)),
        ],
        out_specs=pl.BlockSpec((blk, DOUT), lambda i: (i, 0)),
        out_shape=jax.ShapeDtypeStruct((NPAD, DOUT), jnp.float32),
    )(feats_pad, W, b2)


def _sc_body(y_hbm, eidx_hbm, out_hbm, idx_v, rows0, rows1, acc_v, sem0, sem1):
    c = lax.axis_index("c")
    s = lax.axis_index("s")
    wid = s * NC + c
    base = wid * NPW

    pltpu.sync_copy(eidx_hbm.at[pl.ds(base * DEG, NPW * DEG)], idx_v)

    def gather_start(bi, buf, sem):
        pltpu.async_copy(y_hbm.at[idx_v.at[pl.ds(bi * ROWS, ROWS)]], buf, sem)

    def gather_wait(bi, buf, sem):
        pltpu.make_async_copy(
            y_hbm.at[idx_v.at[pl.ds(bi * ROWS, ROWS)]], buf, sem
        ).wait()

    def process(bi, rows_v):
        def node_body(ni, carry2):
            r0 = ni * DEG
            for ch in range(DOUT // L):
                col = pl.ds(ch * L, L)
                p0 = rows_v[r0 + 0, col]
                p1 = rows_v[r0 + 1, col]
                p2 = rows_v[r0 + 2, col]
                p3 = rows_v[r0 + 3, col]
                for j in range(4, DEG, 4):
                    p0 = p0 + rows_v[r0 + j + 0, col]
                    p1 = p1 + rows_v[r0 + j + 1, col]
                    p2 = p2 + rows_v[r0 + j + 2, col]
                    p3 = p3 + rows_v[r0 + j + 3, col]
                acc_v[bi * NB + ni, col] = (p0 + p1) + (p2 + p3)
            return carry2

        lax.fori_loop(0, NB, node_body, 0)

    gather_start(0, rows0, sem0)

    def pair_body(g, carry):
        bi0 = g * 2
        gather_start(bi0 + 1, rows1, sem1)
        gather_wait(bi0, rows0, sem0)
        process(bi0, rows0)

        @pl.when(g < NBATCH // 2 - 1)
        def _():
            gather_start(bi0 + 2, rows0, sem0)

        gather_wait(bi0 + 1, rows1, sem1)
        process(bi0 + 1, rows1)
        return carry

    lax.fori_loop(0, NBATCH // 2, pair_body, 0)

    pltpu.sync_copy(acc_v, out_hbm.at[pl.ds(base, NPW)])


def _sc_gather(y, eidx):
    mesh = plsc.VectorSubcoreMesh(core_axis_name="c", subcore_axis_name="s")
    fn = functools.partial(
        pl.kernel,
        mesh=mesh,
        out_type=jax.ShapeDtypeStruct((NPAD, DOUT), jnp.float32),
        scratch_types=[
            pltpu.VMEM((NPW * DEG,), jnp.int32),
            pltpu.VMEM((ROWS, DOUT), jnp.float32),
            pltpu.VMEM((ROWS, DOUT), jnp.float32),
            pltpu.VMEM((NPW, DOUT), jnp.float32),
            pltpu.SemaphoreType.DMA,
            pltpu.SemaphoreType.DMA,
        ],
    )(_sc_body)
    return fn(y, eidx)


@jax.jit
def _impl(feats, edge_dict, W, b):
    feats_pad = jnp.pad(feats, ((0, NPAD - N), (0, 0)))
    y = _linear_relu_scaled(feats_pad, W, b.reshape(1, DOUT))
    eidx = jnp.pad(edge_dict, ((0, NPAD - N), (0, 0))).reshape(-1)
    pooled = _sc_gather(y, eidx)
    return pooled[:N]


def kernel(ids, feats, edge_dict, G, ite, W, b):
    return _impl(feats, edge_dict, W, b)


# speedup vs baseline: 1.4651x; 1.0019x over previous
import functools

import jax
import jax.numpy as jnp
from jax import lax
from jax.experimental import pallas as pl
from jax.experimental.pallas import tpu as pltpu
from jax.experimental.pallas import tpu_sc as plsc

N = 10000
DEG = 32
DIN = 128
DOUT = 128

NC = 2
NS = 16
L = 16
NW = NC * NS
NPW = 320
NPAD = NW * NPW
NB = 4
ROWS = NB * DEG
NBATCH = NPW // NB
K = 4


def _mm_body(f_ref, w_ref, b_ref, o_ref):
    y = jnp.dot(f_ref[...], w_ref[...], preferred_element_type=jnp.float32)
    o_ref[...] = jnp.maximum(y + b_ref[...], 0.0) * (1.0 / DEG)


def _linear_relu_scaled(feats_pad, W, b2):
    blk = 2560
    return pl.pallas_call(
        _mm_body,
        grid=(NPAD // blk,),
        in_specs=[
            pl.BlockSpec((blk, DIN), lambda i: (i, 0)),
            pl.BlockSpec((DIN, DOUT), lambda i: (0, 0)),
            pl.BlockSpec((1, DOUT), lambda i: (0, 0)),
        ],
        out_specs=pl.BlockSpec((blk, DOUT), lambda i: (i, 0)),
        out_shape=jax.ShapeDtypeStruct((NPAD, DOUT), jnp.float32),
    )(feats_pad, W, b2)


def _sc_body(y_hbm, eidx_hbm, out_hbm, idx_v, *rest):
    bufs = rest[:K]
    acc_v = rest[K]
    sems = rest[K + 1:]
    c = lax.axis_index("c")
    s = lax.axis_index("s")
    wid = s * NC + c
    base = wid * NPW

    pltpu.sync_copy(eidx_hbm.at[pl.ds(base * DEG, NPW * DEG)], idx_v)

    def gather_start(bi, buf, sem):
        pltpu.async_copy(y_hbm.at[idx_v.at[pl.ds(bi * ROWS, ROWS)]], buf, sem)

    def gather_wait(bi, buf, sem):
        pltpu.make_async_copy(
            y_hbm.at[idx_v.at[pl.ds(bi * ROWS, ROWS)]], buf, sem
        ).wait()

    def process(bi, rows_v):
        def node_body(ni, carry2):
            r0 = ni * DEG
            for ch in range(DOUT // L):
                col = pl.ds(ch * L, L)
                p0 = rows_v[r0 + 0, col]
                p1 = rows_v[r0 + 1, col]
                p2 = rows_v[r0 + 2, col]
                p3 = rows_v[r0 + 3, col]
                for j in range(4, DEG, 4):
                    p0 = p0 + rows_v[r0 + j + 0, col]
                    p1 = p1 + rows_v[r0 + j + 1, col]
                    p2 = p2 + rows_v[r0 + j + 2, col]
                    p3 = p3 + rows_v[r0 + j + 3, col]
                acc_v[bi * NB + ni, col] = (p0 + p1) + (p2 + p3)
            return carry2

        lax.fori_loop(0, NB, node_body, 0)

    for j in range(K):
        gather_start(j, bufs[j], sems[j])

    def group_body(g, carry):
        b0 = g * K
        for j in range(K):
            bi = b0 + j
            gather_wait(bi, bufs[j], sems[j])
            process(bi, bufs[j])

            @pl.when(bi + K < NBATCH)
            def _():
                gather_start(bi + K, bufs[j], sems[j])
        return carry

    lax.fori_loop(0, NBATCH // K, group_body, 0)

    pltpu.sync_copy(acc_v, out_hbm.at[pl.ds(base, NPW)])


def _sc_gather(y, eidx):
    mesh = plsc.VectorSubcoreMesh(core_axis_name="c", subcore_axis_name="s")
    fn = functools.partial(
        pl.kernel,
        mesh=mesh,
        out_type=jax.ShapeDtypeStruct((NPAD, DOUT), jnp.float32),
        scratch_types=(
            [pltpu.VMEM((NPW * DEG,), jnp.int32)]
            + [pltpu.VMEM((ROWS, DOUT), jnp.float32) for _ in range(K)]
            + [pltpu.VMEM((NPW, DOUT), jnp.float32)]
            + [pltpu.SemaphoreType.DMA for _ in range(K)]
        ),
    )(_sc_body)
    return fn(y, eidx)


@jax.jit
def _impl(feats, edge_dict, W, b):
    feats_pad = jnp.pad(feats, ((0, NPAD - N), (0, 0)))
    y = _linear_relu_scaled(feats_pad, W, b.reshape(1, DOUT))
    eidx = jnp.pad(edge_dict, ((0, NPAD - N), (0, 0))).reshape(-1)
    pooled = _sc_gather(y, eidx)
    return pooled[:N]


def kernel(ids, feats, edge_dict, G, ite, W, b):
    return _impl(feats, edge_dict, W, b)


# speedup vs baseline: 2.3000x; 1.5699x over previous
import functools

import jax
import jax.numpy as jnp
from jax import lax
from jax.experimental import pallas as pl
from jax.experimental.pallas import tpu as pltpu
from jax.experimental.pallas import tpu_sc as plsc

N = 10000
DEG = 32
DIN = 128
DOUT = 128

NC = 2
NS = 16
L = 16
NW = NC * NS
NPW = 320
NPAD = NW * NPW
NB = 4
ROWS = NB * DEG
NBATCH = NPW // NB
K = 4


def _mm_body(f_ref, w_ref, b_ref, o_ref):
    y = jnp.dot(f_ref[...], w_ref[...], preferred_element_type=jnp.float32)
    o_ref[...] = (jnp.maximum(y + b_ref[...], 0.0) * (1.0 / DEG)).astype(jnp.bfloat16)


def _linear_relu_scaled(feats_pad, W, b2):
    blk = 2560
    return pl.pallas_call(
        _mm_body,
        grid=(NPAD // blk,),
        in_specs=[
            pl.BlockSpec((blk, DIN), lambda i: (i, 0)),
            pl.BlockSpec((DIN, DOUT), lambda i: (0, 0)),
            pl.BlockSpec((1, DOUT), lambda i: (0, 0)),
        ],
        out_specs=pl.BlockSpec((blk, DOUT), lambda i: (i, 0)),
        out_shape=jax.ShapeDtypeStruct((NPAD, DOUT), jnp.bfloat16),
    )(feats_pad, W, b2)


def _sc_body(y_hbm, eidx_hbm, out_hbm, idx_v, *rest):
    bufs = rest[:K]
    stages = rest[K:2 * K]
    sems = rest[2 * K:]
    c = lax.axis_index("c")
    s = lax.axis_index("s")
    wid = s * NC + c
    base = wid * NPW

    pltpu.sync_copy(eidx_hbm.at[pl.ds(base * DEG, NPW * DEG)], idx_v)

    def gather_start(bi, buf, sem):
        pltpu.async_copy(y_hbm.at[idx_v.at[pl.ds(bi * ROWS, ROWS)]], buf, sem)

    def gather_wait(bi, buf, sem):
        pltpu.make_async_copy(
            y_hbm.at[idx_v.at[pl.ds(bi * ROWS, ROWS)]], buf, sem
        ).wait()

    HI_MASK = jnp.full((L,), -65536, dtype=jnp.int32)
    SH16 = jnp.full((L,), 16, dtype=jnp.int32)

    def load2(rows_v, r, col):
        w = rows_v[r, col]
        a = lax.bitcast_convert_type(jnp.left_shift(w, SH16), jnp.float32)
        b = lax.bitcast_convert_type(jnp.bitwise_and(w, HI_MASK), jnp.float32)
        return a, b

    def process(bi, rows_v, stage_v):
        for ni in range(NB):
            r0 = ni * DEG
            for ch in range(DOUT // 32):
                col = pl.ds(ch * 16, 16)
                a0, b0 = load2(rows_v, r0 + 0, col)
                a1, b1 = load2(rows_v, r0 + 1, col)
                for j in range(2, DEG, 2):
                    x0, y0 = load2(rows_v, r0 + j + 0, col)
                    x1, y1 = load2(rows_v, r0 + j + 1, col)
                    a0 = a0 + x0
                    b0 = b0 + y0
                    a1 = a1 + x1
                    b1 = b1 + y1
                stage_v[ni, pl.ds(ch * 32, 16)] = a0 + a1
                stage_v[ni, pl.ds(ch * 32 + 16, 16)] = b0 + b1
        pltpu.sync_copy(stage_v, out_hbm.at[pl.ds(base + bi * NB, NB)])

    for j in range(K):
        gather_start(j, bufs[j], sems[j])

    def group_body(g, carry):
        b0 = g * K
        for j in range(K):
            bi = b0 + j
            gather_wait(bi, bufs[j], sems[j])
            process(bi, bufs[j], stages[j])

            @pl.when(bi + K < NBATCH)
            def _():
                gather_start(bi + K, bufs[j], sems[j])
        return carry

    lax.fori_loop(0, NBATCH // K, group_body, 0)


def _sc_gather(y, eidx):
    mesh = plsc.VectorSubcoreMesh(core_axis_name="c", subcore_axis_name="s")
    fn = functools.partial(
        pl.kernel,
        mesh=mesh,
        out_type=jax.ShapeDtypeStruct((NPAD, DOUT), jnp.float32),
        compiler_params=pltpu.CompilerParams(use_tc_tiling_on_sc=False),
        scratch_types=(
            [pltpu.VMEM((NPW * DEG,), jnp.int32)]
            + [pltpu.VMEM((ROWS, DOUT // 2), jnp.int32) for _ in range(K)]
            + [pltpu.VMEM((NB, DOUT), jnp.float32) for _ in range(K)]
            + [pltpu.SemaphoreType.DMA for _ in range(K)]
        ),
    )(_sc_body)
    return fn(y, eidx)


@jax.jit
def _impl(feats, edge_dict, W, b):
    feats_pad = jnp.pad(feats, ((0, NPAD - N), (0, 0)))
    y = _linear_relu_scaled(feats_pad, W, b.reshape(1, DOUT))
    y_packed = jax.lax.bitcast_convert_type(
        y.reshape(NPAD, DOUT // 2, 2), jnp.int32)
    eidx = jnp.pad(edge_dict, ((0, NPAD - N), (0, 0))).reshape(-1)
    pooled = _sc_gather(y_packed, eidx)
    return (pooled[:N].reshape(N, DOUT // 32, 2, 16)
            .transpose(0, 1, 3, 2).reshape(N, DOUT))


def kernel(ids, feats, edge_dict, G, ite, W, b):
    return _impl(feats, edge_dict, W, b)


# speedup vs baseline: 4.2252x; 1.8370x over previous
import functools

import jax
import jax.numpy as jnp
from jax import lax
from jax.experimental import pallas as pl
from jax.experimental.pallas import tpu as pltpu
from jax.experimental.pallas import tpu_sc as plsc

N = 10000
DEG = 32
DIN = 128
DOUT = 128

NC = 2
NS = 16
L = 16
NW = NC * NS
NPW = 320
NPAD = NW * NPW
NB = 4
ROWS = NB * DEG
NBATCH = NPW // NB
K = 4


def _mm_body(f_ref, w_ref, b_ref, o_ref):
    y = jnp.dot(f_ref[...], w_ref[...], preferred_element_type=jnp.float32)
    o_ref[...] = (jnp.maximum(y + b_ref[...], 0.0) * (1.0 / DEG)).astype(jnp.bfloat16)


def _linear_relu_scaled(feats_pad, W, b2):
    blk = 2560
    return pl.pallas_call(
        _mm_body,
        grid=(NPAD // blk,),
        in_specs=[
            pl.BlockSpec((blk, DIN), lambda i: (i, 0)),
            pl.BlockSpec((DIN, DOUT), lambda i: (0, 0)),
            pl.BlockSpec((1, DOUT), lambda i: (0, 0)),
        ],
        out_specs=pl.BlockSpec((blk, DOUT), lambda i: (i, 0)),
        out_shape=jax.ShapeDtypeStruct((NPAD, DOUT), jnp.bfloat16),
    )(feats_pad, W, b2)


def _sc_body(y_hbm, eidx_hbm, out_hbm, idx_v, y_sh, *rest):
    bufs = rest[:K]
    stages = rest[K:2 * K]
    sems = rest[2 * K:]
    c = lax.axis_index("c")
    s = lax.axis_index("s")
    wid = s * NC + c
    base = wid * NPW

    rpt = NPAD // NS
    pltpu.sync_copy(y_hbm.at[pl.ds(s * rpt, rpt)], y_sh.at[pl.ds(s * rpt, rpt)])

    pltpu.sync_copy(eidx_hbm.at[pl.ds(base * DEG, NPW * DEG)], idx_v)
    plsc.subcore_barrier()

    def gather_start(bi, buf, sem):
        pltpu.async_copy(y_sh.at[idx_v.at[pl.ds(bi * ROWS, ROWS)]], buf, sem)

    def gather_wait(bi, buf, sem):
        pltpu.make_async_copy(
            y_sh.at[idx_v.at[pl.ds(bi * ROWS, ROWS)]], buf, sem
        ).wait()

    HI_MASK = jnp.full((L,), -65536, dtype=jnp.int32)
    SH16 = jnp.full((L,), 16, dtype=jnp.int32)

    def load2(rows_v, r, col):
        w = rows_v[r, col]
        a = lax.bitcast_convert_type(jnp.left_shift(w, SH16), jnp.float32)
        b = lax.bitcast_convert_type(jnp.bitwise_and(w, HI_MASK), jnp.float32)
        return a, b

    def process(bi, rows_v, stage_v):
        for ni in range(NB):
            r0 = ni * DEG
            for ch in range(DOUT // 32):
                col = pl.ds(ch * 16, 16)
                a0, b0 = load2(rows_v, r0 + 0, col)
                a1, b1 = load2(rows_v, r0 + 1, col)
                for j in range(2, DEG, 2):
                    x0, y0 = load2(rows_v, r0 + j + 0, col)
                    x1, y1 = load2(rows_v, r0 + j + 1, col)
                    a0 = a0 + x0
                    b0 = b0 + y0
                    a1 = a1 + x1
                    b1 = b1 + y1
                stage_v[ni, pl.ds(ch * 32, 16)] = a0 + a1
                stage_v[ni, pl.ds(ch * 32 + 16, 16)] = b0 + b1
        pltpu.sync_copy(stage_v, out_hbm.at[pl.ds(base + bi * NB, NB)])

    for j in range(K):
        gather_start(j, bufs[j], sems[j])

    def group_body(g, carry):
        b0 = g * K
        for j in range(K):
            bi = b0 + j
            gather_wait(bi, bufs[j], sems[j])
            process(bi, bufs[j], stages[j])

            @pl.when(bi + K < NBATCH)
            def _():
                gather_start(bi + K, bufs[j], sems[j])
        return carry

    lax.fori_loop(0, NBATCH // K, group_body, 0)


def _sc_gather(y, eidx):
    mesh = plsc.VectorSubcoreMesh(core_axis_name="c", subcore_axis_name="s")
    fn = functools.partial(
        pl.kernel,
        mesh=mesh,
        out_type=jax.ShapeDtypeStruct((NPAD, DOUT), jnp.float32),
        compiler_params=pltpu.CompilerParams(use_tc_tiling_on_sc=False),
        scratch_types=(
            [pltpu.VMEM((NPW * DEG,), jnp.int32)]
            + [pltpu.VMEM_SHARED((NPAD, DOUT // 2), jnp.int32)]
            + [pltpu.VMEM((ROWS, DOUT // 2), jnp.int32) for _ in range(K)]
            + [pltpu.VMEM((NB, DOUT), jnp.float32) for _ in range(K)]
            + [pltpu.SemaphoreType.DMA for _ in range(K)]
        ),
    )(_sc_body)
    return fn(y, eidx)


@jax.jit
def _impl(feats, edge_dict, W, b):
    feats_pad = jnp.pad(feats, ((0, NPAD - N), (0, 0)))
    y = _linear_relu_scaled(feats_pad, W, b.reshape(1, DOUT))
    y_packed = jax.lax.bitcast_convert_type(
        y.reshape(NPAD, DOUT // 2, 2), jnp.int32)
    eidx = jnp.pad(edge_dict, ((0, NPAD - N), (0, 0))).reshape(-1)
    pooled = _sc_gather(y_packed, eidx)
    return (pooled[:N].reshape(N, DOUT // 32, 2, 16)
            .transpose(0, 1, 3, 2).reshape(N, DOUT))


def kernel(ids, feats, edge_dict, G, ite, W, b):
    return _impl(feats, edge_dict, W, b)


# speedup vs baseline: 7.4275x; 1.7579x over previous
import functools

import jax
import jax.numpy as jnp
from jax import lax
from jax.experimental import pallas as pl
from jax.experimental.pallas import tpu as pltpu
from jax.experimental.pallas import tpu_sc as plsc

N = 10000
DEG = 32
DIN = 128
DOUT = 128
HALF = DOUT // 2

NC = 2
NS = 16
L = 16
NW = NC * NS
NPW = 320
NPAD = NW * NPW
NB = 4
ROWS = NB * DEG
NBATCH = NPW // NB
K = 4


def _mm_body(f_ref, w_ref, b_ref, o_ref):
    y = jnp.dot(f_ref[...], w_ref[...], preferred_element_type=jnp.float32)
    y = jnp.maximum(y + b_ref[...], 0.0) * (1.0 / DEG)
    a_bits = lax.bitcast_convert_type(y[:, :HALF], jnp.int32)
    b_bits = lax.bitcast_convert_type(y[:, HALF:], jnp.int32)
    a_r = a_bits + 0x7FFF + jnp.bitwise_and(jnp.right_shift(a_bits, 16), 1)
    b_r = b_bits + 0x7FFF + jnp.bitwise_and(jnp.right_shift(b_bits, 16), 1)
    a_top = jnp.bitwise_and(a_r, jnp.int32(-65536))
    b_top = jnp.bitwise_and(jnp.right_shift(b_r, 16), jnp.int32(0xFFFF))
    o_ref[...] = jnp.bitwise_or(a_top, b_top)


def _linear_relu_pack(feats, W, b2):
    blk = 2560
    return pl.pallas_call(
        _mm_body,
        grid=(NPAD // blk,),
        in_specs=[
            pl.BlockSpec((blk, DIN), lambda i: (i, 0)),
            pl.BlockSpec((DIN, DOUT), lambda i: (0, 0)),
            pl.BlockSpec((1, DOUT), lambda i: (0, 0)),
        ],
        out_specs=pl.BlockSpec((blk, HALF), lambda i: (i, 0)),
        out_shape=jax.ShapeDtypeStruct((NPAD, HALF), jnp.int32),
    )(feats, W, b2)


def _sc_body(y_hbm, eidx_hbm, out_hbm, idx_v, y_sh, *rest):
    bufs = rest[:K]
    stages = rest[K:2 * K]
    sems = rest[2 * K:]
    c = lax.axis_index("c")
    s = lax.axis_index("s")
    wid = s * NC + c
    base = wid * NPW

    rpt = NPAD // NS
    pltpu.sync_copy(y_hbm.at[pl.ds(s * rpt, rpt)], y_sh.at[pl.ds(s * rpt, rpt)])

    pltpu.sync_copy(eidx_hbm.at[pl.ds(base * DEG, NPW * DEG)], idx_v)
    plsc.subcore_barrier()

    def gather_start(bi, buf, sem):
        pltpu.async_copy(y_sh.at[idx_v.at[pl.ds(bi * ROWS, ROWS)]], buf, sem)

    def gather_wait(bi, buf, sem):
        pltpu.make_async_copy(
            y_sh.at[idx_v.at[pl.ds(bi * ROWS, ROWS)]], buf, sem
        ).wait()

    SH16 = jnp.full((L,), 16, dtype=jnp.int32)

    def load2(rows_v, r, col):
        w = rows_v[r, col]
        a = lax.bitcast_convert_type(w, jnp.float32)
        b = lax.bitcast_convert_type(jnp.left_shift(w, SH16), jnp.float32)
        return a, b

    def process(bi, rows_v, stage_v):
        for ni in range(NB):
            r0 = ni * DEG
            for ch in range(HALF // L):
                col = pl.ds(ch * L, L)
                a0, b0 = load2(rows_v, r0 + 0, col)
                a1, b1 = load2(rows_v, r0 + 1, col)
                for j in range(2, DEG, 2):
                    x0, y0 = load2(rows_v, r0 + j + 0, col)
                    x1, y1 = load2(rows_v, r0 + j + 1, col)
                    a0 = a0 + x0
                    b0 = b0 + y0
                    a1 = a1 + x1
                    b1 = b1 + y1
                stage_v[ni, pl.ds(ch * L, L)] = a0 + a1
                stage_v[ni, pl.ds(HALF + ch * L, L)] = b0 + b1
        @pl.when(base + bi * NB + NB <= N)
        def _():
            pltpu.sync_copy(stage_v, out_hbm.at[pl.ds(base + bi * NB, NB)])

    for j in range(K):
        gather_start(j, bufs[j], sems[j])

    def group_body(g, carry):
        b0 = g * K
        for j in range(K):
            bi = b0 + j
            gather_wait(bi, bufs[j], sems[j])
            process(bi, bufs[j], stages[j])

            @pl.when(bi + K < NBATCH)
            def _():
                gather_start(bi + K, bufs[j], sems[j])
        return carry

    lax.fori_loop(0, NBATCH // K, group_body, 0)


def _sc_gather(y_packed, eidx):
    mesh = plsc.VectorSubcoreMesh(core_axis_name="c", subcore_axis_name="s")
    fn = functools.partial(
        pl.kernel,
        mesh=mesh,
        out_type=jax.ShapeDtypeStruct((N, DOUT), jnp.float32),
        compiler_params=pltpu.CompilerParams(use_tc_tiling_on_sc=False),
        scratch_types=(
            [pltpu.VMEM((NPW * DEG,), jnp.int32)]
            + [pltpu.VMEM_SHARED((NPAD, HALF), jnp.int32)]
            + [pltpu.VMEM((ROWS, HALF), jnp.int32) for _ in range(K)]
            + [pltpu.VMEM((NB, DOUT), jnp.float32) for _ in range(K)]
            + [pltpu.SemaphoreType.DMA for _ in range(K)]
        ),
    )(_sc_body)
    return fn(y_packed, eidx)


@jax.jit
def _impl(feats, edge_dict, W, b):
    y_packed = _linear_relu_pack(feats, W, b.reshape(1, DOUT))
    eidx = jnp.pad(edge_dict.reshape(-1), (0, (NPAD - N) * DEG))
    return _sc_gather(y_packed, eidx)


def kernel(ids, feats, edge_dict, G, ite, W, b):
    return _impl(feats, edge_dict, W, b)
